# baseline (speedup 1.0000x reference)
import functools

import jax
import jax.numpy as jnp
from jax import lax
from jax.experimental import pallas as pl
from jax.experimental.pallas import tpu as pltpu
from jax.experimental.pallas import tpu_sc as plsc

N_EMBD = 1024
N_EXP = 8
TOP_K = 2
B, T = 1, 2048
N_TOK = B * T
CAP = int(TOP_K * 1.25 * N_TOK / N_EXP)
CAP += CAP % 2
NSLOT = N_EXP * CAP
N_ENT = TOP_K * N_TOK
HID = 4 * N_EMBD
SENT_TOK = N_TOK

NC, NS = 2, 16
NW = NC * NS
NG = 2
EG = N_EXP // NG
GSLOT = NSLOT // NG
ROWS_PER_W = GSLOT // NW
DISP_CHUNK = 40
TOK_PER_W = N_TOK // NW
COMB_CHUNK = 32


def _router_body(x_ref, wg_ref, ent_ref, wgt_ref):
    x = x_ref[...]
    wg = wg_ref[...]
    logits = jnp.dot(x, wg, preferred_element_type=jnp.float32)
    iota_e = lax.broadcasted_iota(jnp.int32, (N_TOK, N_EXP), 1)
    m1 = jnp.max(logits, axis=1, keepdims=True)
    e1 = jnp.min(jnp.where(logits == m1, iota_e, N_EXP), axis=1, keepdims=True)
    masked2 = jnp.where(iota_e == e1, -jnp.inf, logits)
    m2 = jnp.max(masked2, axis=1, keepdims=True)
    e2 = jnp.min(jnp.where(masked2 == m2, iota_e, N_EXP), axis=1, keepdims=True)
    z = jnp.exp(m2 - m1)
    p1 = 1.0 / (1.0 + z)
    p2 = z / (1.0 + z)

    ek = jnp.concatenate([e1, e2], axis=0)
    pk = jnp.concatenate([p1, p2], axis=0)
    oh = (ek == lax.broadcasted_iota(jnp.int32, (N_ENT, N_EXP), 1)).astype(
        jnp.float32)
    c = oh
    s = 1
    while s < N_ENT:
        c = c + jnp.concatenate(
            [jnp.zeros((s, N_EXP), jnp.float32), c[: N_ENT - s]], axis=0)
        s *= 2
    rank = jnp.sum(c * oh, axis=1, keepdims=True) - 1.0
    rank_i = rank.astype(jnp.int32)
    keep = rank_i < CAP
    slot = ek * CAP + rank_i
    slot_d = jnp.where(keep, slot, NSLOT)
    w = jnp.where(keep, pk, 0.0)
    ent_ref[...] = jnp.broadcast_to(slot_d, (N_ENT, N_EXP))
    wgt_ref[...] = jnp.broadcast_to(w, (N_ENT, N_EXP))


def _router(x2d, w_g):
    return pl.pallas_call(
        _router_body,
        out_shape=[
            jax.ShapeDtypeStruct((N_ENT, N_EXP), jnp.int32),
            jax.ShapeDtypeStruct((N_ENT, N_EXP), jnp.float32),
        ],
    )(x2d, w_g)


def _dispatch_body(xpad_hbm, ent_hbm, out_hbm, tbl, entv, rows0, rows1,
                   gsem, ssem, *, gbase):
    wid = lax.axis_index("s") * NC + lax.axis_index("c")
    base = gbase + wid * ROWS_PER_W
    obase = wid * ROWS_PER_W
    for i in range((ROWS_PER_W + 15) // 16):
        tbl[pl.ds(i * 16, 16)] = jnp.zeros((16,), jnp.int32)
    pltpu.sync_copy(ent_hbm, entv)
    def scat(i, _):
        for u in range(4):
            ii = i * 4 + u
            s = entv[pl.ds(ii * 16, 16)]
            j = ii * 16 + lax.iota(jnp.int32, 16)
            tok = jnp.bitwise_and(j, N_TOK - 1)
            loc = s - base
            msk = (loc >= 0) & (loc < ROWS_PER_W)
            plsc.store_scatter(tbl, [loc], tok, mask=msk)
        return 0
    lax.fori_loop(0, N_ENT // 64, scat, 0)
    bufs = (rows0, rows1)
    nch = ROWS_PER_W // DISP_CHUNK
    stores = [None] * nch
    for cch in range(nch):
        idx = tbl.at[pl.ds(cch * DISP_CHUNK, DISP_CHUNK)]
        buf = bufs[cch % 2]
        if cch >= 2:
            stores[cch - 2].wait()
        pltpu.async_copy(xpad_hbm.at[idx], buf, gsem).wait()
        stores[cch] = pltpu.async_copy(
            buf, out_hbm.at[pl.ds(obase + cch * DISP_CHUNK, DISP_CHUNK)], ssem)
    for cch in range(max(0, nch - 2), nch):
        stores[cch].wait()


def _dispatch(xpad, ent, g):
    mesh = plsc.VectorSubcoreMesh(core_axis_name="c", subcore_axis_name="s")
    f = functools.partial(
        pl.kernel,
        out_type=jax.ShapeDtypeStruct((GSLOT, N_EMBD), jnp.float32),
        mesh=mesh,
        scratch_types=[
            pltpu.VMEM((256,), jnp.int32),
            pltpu.VMEM((N_ENT,), jnp.int32),
            pltpu.VMEM((DISP_CHUNK, N_EMBD), jnp.float32),
            pltpu.VMEM((DISP_CHUNK, N_EMBD), jnp.float32),
            pltpu.SemaphoreType.DMA,
            pltpu.SemaphoreType.DMA,
        ],
        compiler_params=pltpu.CompilerParams(needs_layout_passes=False),
    )(functools.partial(_dispatch_body, gbase=g * GSLOT))
    return f(xpad, ent)


HBLK = 2048
NH = HID // HBLK


def _mlp_body(x_ref, fc_ref, pj_ref, *rest):
    out_ref = rest[-1]
    hid = pl.program_id(1)
    xe = x_ref[0].astype(jnp.bfloat16)
    h = jnp.dot(xe, fc_ref[0].astype(jnp.bfloat16),
                preferred_element_type=jnp.float32)
    h = 0.5 * h * (1.0 + lax.erf(h * (2.0 ** -0.5)))
    part = jnp.dot(h.astype(jnp.bfloat16), pj_ref[0].astype(jnp.bfloat16),
                   preferred_element_type=jnp.float32)

    @pl.when(hid == 0)
    def _():
        out_ref[...] = part[None]

    @pl.when(hid != 0)
    def _():
        out_ref[...] = out_ref[...] + part[None]


def _mlp(expx_g, c_fc, c_proj, goff):
    return pl.pallas_call(
        _mlp_body,
        grid=(EG, NH),
        in_specs=[
            pl.BlockSpec((1, CAP, N_EMBD), lambda e, h: (e, 0, 0)),
            pl.BlockSpec((1, N_EMBD, HBLK), lambda e, h: (e + goff, 0, h)),
            pl.BlockSpec((1, HBLK, N_EMBD), lambda e, h: (e + goff, h, 0)),
        ],
        out_specs=pl.BlockSpec((1, CAP, N_EMBD), lambda e, h: (e, 0, 0)),
        out_shape=jax.ShapeDtypeStruct((EG, CAP, N_EMBD), jnp.float32),
        compiler_params=pltpu.CompilerParams(
            dimension_semantics=("parallel", "arbitrary")),
    )(expx_g, c_fc, c_proj)


def _combine_body(*refs, gbase, has_prev):
    if has_prev:
        (y_hbm, ent_hbm, wgt_hbm, prev_hbm, out_hbm,
         s0v, s1v, w0v, w1v, av, bv, pv, sem) = refs
    else:
        (y_hbm, ent_hbm, wgt_hbm, out_hbm,
         s0v, s1v, w0v, w1v, av, bv, pv, sem) = refs
    wid = lax.axis_index("s") * NC + lax.axis_index("c")
    tb = wid * TOK_PER_W
    for cch in range(TOK_PER_W // COMB_CHUNK):
        t0 = tb + cch * COMB_CHUNK
        pltpu.sync_copy(ent_hbm.at[pl.ds(t0, COMB_CHUNK)], s0v)
        pltpu.sync_copy(ent_hbm.at[pl.ds(N_TOK + t0, COMB_CHUNK)], s1v)
        pltpu.sync_copy(wgt_hbm.at[pl.ds(t0, COMB_CHUNK)], w0v)
        pltpu.sync_copy(wgt_hbm.at[pl.ds(N_TOK + t0, COMB_CHUNK)], w1v)
        for i in range(COMB_CHUNK // 16):
            sl = pl.ds(i * 16, 16)
            l0 = s0v[sl] - gbase
            in0 = (l0 >= 0) & (l0 < GSLOT)
            s0v[sl] = jnp.minimum(jnp.maximum(l0, 0), GSLOT - 1)
            w0v[sl] = jnp.where(in0, w0v[sl], 0.0)
            l1 = s1v[sl] - gbase
            in1 = (l1 >= 0) & (l1 < GSLOT)
            s1v[sl] = jnp.minimum(jnp.maximum(l1, 0), GSLOT - 1)
            w1v[sl] = jnp.where(in1, w1v[sl], 0.0)
        cpa = pltpu.async_copy(y_hbm.at[s0v], av, sem)
        cpb = pltpu.async_copy(y_hbm.at[s1v], bv, sem)
        if has_prev:
            cpp = pltpu.async_copy(
                prev_hbm.at[pl.ds(t0, COMB_CHUNK)], pv, sem)
        cpa.wait()
        cpb.wait()
        if has_prev:
            cpp.wait()

        for i16 in range(COMB_CHUNK // 16):
            w0c = w0v[pl.ds(i16 * 16, 16)]
            w1c = w1v[pl.ds(i16 * 16, 16)]
            for rr in range(16):
                r = i16 * 16 + rr
                w0 = w0c[rr]
                w1 = w1c[rr]

                def col(j, _, r=r, w0=w0, w1=w1):
                    for u in range(16):
                        cs = pl.ds((j * 16 + u) * 16, 16)
                        acc = w0 * av[r, cs] + w1 * bv[r, cs]
                        if has_prev:
                            acc = acc + pv[r, cs]
                        av[r, cs] = acc
                    return 0
                lax.fori_loop(0, N_EMBD // 256, col, 0)
        pltpu.sync_copy(av, out_hbm.at[pl.ds(t0, COMB_CHUNK)])


def _combine(y2d_g, ent, wgt, prev, g):
    mesh = plsc.VectorSubcoreMesh(core_axis_name="c", subcore_axis_name="s")
    body = functools.partial(
        _combine_body, gbase=g * GSLOT, has_prev=prev is not None)
    f = functools.partial(
        pl.kernel,
        out_type=jax.ShapeDtypeStruct((N_TOK, N_EMBD), jnp.float32),
        mesh=mesh,
        scratch_types=[
            pltpu.VMEM((COMB_CHUNK,), jnp.int32),
            pltpu.VMEM((COMB_CHUNK,), jnp.int32),
            pltpu.VMEM((COMB_CHUNK,), jnp.float32),
            pltpu.VMEM((COMB_CHUNK,), jnp.float32),
            pltpu.VMEM((COMB_CHUNK, N_EMBD), jnp.float32),
            pltpu.VMEM((COMB_CHUNK, N_EMBD), jnp.float32),
            pltpu.VMEM((COMB_CHUNK, N_EMBD), jnp.float32),
            pltpu.SemaphoreType.DMA,
        ],
        compiler_params=pltpu.CompilerParams(needs_layout_passes=False),
    )(body)
    if prev is not None:
        return f(y2d_g, ent, wgt, prev)
    return f(y2d_g, ent, wgt)


def kernel(x, w_g, c_fc, c_proj):
    x2d = x.reshape(N_TOK, N_EMBD)
    ent8, wgt8 = _router(x2d, w_g)
    ent = ent8[:, 0]
    wgt = wgt8[:, 0]
    ys = []
    for g in range(NG):
        expx_g = _dispatch(x2d, ent, g)
        ys.append(_mlp(expx_g.reshape(EG, CAP, N_EMBD), c_fc, c_proj, g * EG))
    out2d = None
    for g in range(NG):
        out2d = _combine(ys[g].reshape(GSLOT, N_EMBD), ent, wgt, out2d, g)
    return out2d.reshape(B, T, N_EMBD)


# speedup vs baseline: 1.7611x; 1.7611x over previous
import functools

import jax
import jax.numpy as jnp
from jax import lax
from jax.experimental import pallas as pl
from jax.experimental.pallas import tpu as pltpu
from jax.experimental.pallas import tpu_sc as plsc

N_EMBD = 1024
N_EXP = 8
TOP_K = 2
B, T = 1, 2048
N_TOK = B * T
CAP = int(TOP_K * 1.25 * N_TOK / N_EXP)
CAP += CAP % 2
NSLOT = N_EXP * CAP
N_ENT = TOP_K * N_TOK
HID = 4 * N_EMBD
SENT_TOK = N_TOK

NC, NS = 2, 16
NW = NC * NS
NG = 1
EG = N_EXP // NG
GSLOT = NSLOT // NG
ROWS_PER_W = GSLOT // NW
DISP_CHUNK = 40
TOK_PER_W = N_TOK // NW
COMB_CHUNK = 32


def _router_body(x_ref, wg_ref, ent_ref, wgt_ref):
    x = x_ref[...]
    wg = wg_ref[...]
    logits = jnp.dot(x, wg, preferred_element_type=jnp.float32)
    iota_e = lax.broadcasted_iota(jnp.int32, (N_TOK, N_EXP), 1)
    m1 = jnp.max(logits, axis=1, keepdims=True)
    e1 = jnp.min(jnp.where(logits == m1, iota_e, N_EXP), axis=1, keepdims=True)
    masked2 = jnp.where(iota_e == e1, -jnp.inf, logits)
    m2 = jnp.max(masked2, axis=1, keepdims=True)
    e2 = jnp.min(jnp.where(masked2 == m2, iota_e, N_EXP), axis=1, keepdims=True)
    z = jnp.exp(m2 - m1)
    p1 = 1.0 / (1.0 + z)
    p2 = z / (1.0 + z)

    ek = jnp.concatenate([e1, e2], axis=0)
    pk = jnp.concatenate([p1, p2], axis=0)
    oh = (ek == lax.broadcasted_iota(jnp.int32, (N_ENT, N_EXP), 1)).astype(
        jnp.float32)
    c = oh
    s = 1
    while s < N_ENT:
        c = c + jnp.concatenate(
            [jnp.zeros((s, N_EXP), jnp.float32), c[: N_ENT - s]], axis=0)
        s *= 2
    rank = jnp.sum(c * oh, axis=1, keepdims=True) - 1.0
    rank_i = rank.astype(jnp.int32)
    keep = rank_i < CAP
    slot = ek * CAP + rank_i
    slot_d = jnp.where(keep, slot, NSLOT)
    w = jnp.where(keep, pk, 0.0)
    ent_ref[...] = jnp.broadcast_to(slot_d, (N_ENT, N_EXP))
    wgt_ref[...] = jnp.broadcast_to(w, (N_ENT, N_EXP))


def _router(x2d, w_g):
    return pl.pallas_call(
        _router_body,
        out_shape=[
            jax.ShapeDtypeStruct((N_ENT, N_EXP), jnp.int32),
            jax.ShapeDtypeStruct((N_ENT, N_EXP), jnp.float32),
        ],
    )(x2d, w_g)


def _dispatch_body(xpad_hbm, ent_hbm, out_hbm, tbl, entv, rows0, rows1,
                   gsem, ssem, *, gbase):
    wid = lax.axis_index("s") * NC + lax.axis_index("c")
    base = gbase + wid * ROWS_PER_W
    obase = wid * ROWS_PER_W
    for i in range((ROWS_PER_W + 15) // 16):
        tbl[pl.ds(i * 16, 16)] = jnp.zeros((16,), jnp.int32)
    pltpu.sync_copy(ent_hbm, entv)
    def scat(i, _):
        for u in range(4):
            ii = i * 4 + u
            s = entv[pl.ds(ii * 16, 16)]
            j = ii * 16 + lax.iota(jnp.int32, 16)
            tok = jnp.bitwise_and(j, N_TOK - 1)
            loc = s - base
            msk = (loc >= 0) & (loc < ROWS_PER_W)
            plsc.store_scatter(tbl, [loc], tok, mask=msk)
        return 0
    lax.fori_loop(0, N_ENT // 64, scat, 0)
    bufs = (rows0, rows1)
    nch = ROWS_PER_W // DISP_CHUNK
    stores = [None] * nch
    for cch in range(nch):
        idx = tbl.at[pl.ds(cch * DISP_CHUNK, DISP_CHUNK)]
        buf = bufs[cch % 2]
        if cch >= 2:
            stores[cch - 2].wait()
        pltpu.async_copy(xpad_hbm.at[idx], buf, gsem).wait()
        stores[cch] = pltpu.async_copy(
            buf, out_hbm.at[pl.ds(obase + cch * DISP_CHUNK, DISP_CHUNK)], ssem)
    for cch in range(max(0, nch - 2), nch):
        stores[cch].wait()


def _dispatch(xpad, ent, g):
    mesh = plsc.VectorSubcoreMesh(core_axis_name="c", subcore_axis_name="s")
    f = functools.partial(
        pl.kernel,
        out_type=jax.ShapeDtypeStruct((GSLOT, N_EMBD), jnp.float32),
        mesh=mesh,
        scratch_types=[
            pltpu.VMEM((256,), jnp.int32),
            pltpu.VMEM((N_ENT,), jnp.int32),
            pltpu.VMEM((DISP_CHUNK, N_EMBD), jnp.float32),
            pltpu.VMEM((DISP_CHUNK, N_EMBD), jnp.float32),
            pltpu.SemaphoreType.DMA,
            pltpu.SemaphoreType.DMA,
        ],
        compiler_params=pltpu.CompilerParams(needs_layout_passes=False),
    )(functools.partial(_dispatch_body, gbase=g * GSLOT))
    return f(xpad, ent)


HBLK = 2048
NH = HID // HBLK


def _mlp_body(x_ref, fc_ref, pj_ref, *rest):
    out_ref = rest[-1]
    hid = pl.program_id(1)
    xe = x_ref[0].astype(jnp.bfloat16)
    h = jnp.dot(xe, fc_ref[0].astype(jnp.bfloat16),
                preferred_element_type=jnp.float32)
    h = 0.5 * h * (1.0 + lax.erf(h * (2.0 ** -0.5)))
    part = jnp.dot(h.astype(jnp.bfloat16), pj_ref[0].astype(jnp.bfloat16),
                   preferred_element_type=jnp.float32)

    @pl.when(hid == 0)
    def _():
        out_ref[...] = part[None]

    @pl.when(hid != 0)
    def _():
        out_ref[...] = out_ref[...] + part[None]


def _mlp(expx_g, c_fc, c_proj, goff):
    return pl.pallas_call(
        _mlp_body,
        grid=(EG, NH),
        in_specs=[
            pl.BlockSpec((1, CAP, N_EMBD), lambda e, h: (e, 0, 0)),
            pl.BlockSpec((1, N_EMBD, HBLK), lambda e, h: (e + goff, 0, h)),
            pl.BlockSpec((1, HBLK, N_EMBD), lambda e, h: (e + goff, h, 0)),
        ],
        out_specs=pl.BlockSpec((1, CAP, N_EMBD), lambda e, h: (e, 0, 0)),
        out_shape=jax.ShapeDtypeStruct((EG, CAP, N_EMBD), jnp.float32),
        compiler_params=pltpu.CompilerParams(
            dimension_semantics=("parallel", "arbitrary")),
    )(expx_g, c_fc, c_proj)


def _combine_body(*refs, gbase, has_prev):
    if has_prev:
        (y_hbm, ent_hbm, wgt_hbm, prev_hbm, out_hbm,
         s0v, s1v, w0v, w1v, av, bv, pv, sem) = refs
    else:
        (y_hbm, ent_hbm, wgt_hbm, out_hbm,
         s0v, s1v, w0v, w1v, av, bv, pv, sem) = refs
    wid = lax.axis_index("s") * NC + lax.axis_index("c")
    tb = wid * TOK_PER_W
    for cch in range(TOK_PER_W // COMB_CHUNK):
        t0 = tb + cch * COMB_CHUNK
        pltpu.sync_copy(ent_hbm.at[pl.ds(t0, COMB_CHUNK)], s0v)
        pltpu.sync_copy(ent_hbm.at[pl.ds(N_TOK + t0, COMB_CHUNK)], s1v)
        pltpu.sync_copy(wgt_hbm.at[pl.ds(t0, COMB_CHUNK)], w0v)
        pltpu.sync_copy(wgt_hbm.at[pl.ds(N_TOK + t0, COMB_CHUNK)], w1v)
        for i in range(COMB_CHUNK // 16):
            sl = pl.ds(i * 16, 16)
            l0 = s0v[sl] - gbase
            in0 = (l0 >= 0) & (l0 < GSLOT)
            s0v[sl] = jnp.minimum(jnp.maximum(l0, 0), GSLOT - 1)
            w0v[sl] = jnp.where(in0, w0v[sl], 0.0)
            l1 = s1v[sl] - gbase
            in1 = (l1 >= 0) & (l1 < GSLOT)
            s1v[sl] = jnp.minimum(jnp.maximum(l1, 0), GSLOT - 1)
            w1v[sl] = jnp.where(in1, w1v[sl], 0.0)
        cpa = pltpu.async_copy(y_hbm.at[s0v], av, sem)
        cpb = pltpu.async_copy(y_hbm.at[s1v], bv, sem)
        if has_prev:
            cpp = pltpu.async_copy(
                prev_hbm.at[pl.ds(t0, COMB_CHUNK)], pv, sem)
        cpa.wait()
        cpb.wait()
        if has_prev:
            cpp.wait()

        for i16 in range(COMB_CHUNK // 16):
            w0c = w0v[pl.ds(i16 * 16, 16)]
            w1c = w1v[pl.ds(i16 * 16, 16)]
            for rr in range(16):
                r = i16 * 16 + rr
                w0 = w0c[rr]
                w1 = w1c[rr]

                def col(j, _, r=r, w0=w0, w1=w1):
                    for u in range(16):
                        cs = pl.ds((j * 16 + u) * 16, 16)
                        acc = w0 * av[r, cs] + w1 * bv[r, cs]
                        if has_prev:
                            acc = acc + pv[r, cs]
                        av[r, cs] = acc
                    return 0
                lax.fori_loop(0, N_EMBD // 256, col, 0)
        pltpu.sync_copy(av, out_hbm.at[pl.ds(t0, COMB_CHUNK)])


def _combine(y2d_g, ent, wgt, prev, g):
    mesh = plsc.VectorSubcoreMesh(core_axis_name="c", subcore_axis_name="s")
    body = functools.partial(
        _combine_body, gbase=g * GSLOT, has_prev=prev is not None)
    f = functools.partial(
        pl.kernel,
        out_type=jax.ShapeDtypeStruct((N_TOK, N_EMBD), jnp.float32),
        mesh=mesh,
        scratch_types=[
            pltpu.VMEM((COMB_CHUNK,), jnp.int32),
            pltpu.VMEM((COMB_CHUNK,), jnp.int32),
            pltpu.VMEM((COMB_CHUNK,), jnp.float32),
            pltpu.VMEM((COMB_CHUNK,), jnp.float32),
            pltpu.VMEM((COMB_CHUNK, N_EMBD), jnp.float32),
            pltpu.VMEM((COMB_CHUNK, N_EMBD), jnp.float32),
            pltpu.VMEM((COMB_CHUNK, N_EMBD), jnp.float32),
            pltpu.SemaphoreType.DMA,
        ],
        compiler_params=pltpu.CompilerParams(needs_layout_passes=False),
    )(body)
    if prev is not None:
        return f(y2d_g, ent, wgt, prev)
    return f(y2d_g, ent, wgt)


def kernel(x, w_g, c_fc, c_proj):
    x2d = x.reshape(N_TOK, N_EMBD)
    ent8, wgt8 = _router(x2d, w_g)
    ent = ent8[:, 0]
    wgt = wgt8[:, 0]
    ys = []
    for g in range(NG):
        expx_g = _dispatch(x2d, ent, g)
        ys.append(_mlp(expx_g.reshape(EG, CAP, N_EMBD), c_fc, c_proj, g * EG))
    out2d = None
    for g in range(NG):
        out2d = _combine(ys[g].reshape(GSLOT, N_EMBD), ent, wgt, out2d, g)
    return out2d.reshape(B, T, N_EMBD)
